# baseline (speedup 1.0000x reference)
import functools

import jax
import jax.numpy as jnp
from jax import lax
from jax.experimental import pallas as pl
from jax.experimental.pallas import tpu as pltpu
from jax.experimental.pallas import tpu_sc as plsc

N = 10000
E = 320000
D = 128
H = 128
G = 64
TEMP = 2.0
EPS = 1e-5
ISQ = float(1.0 / (1.0 + EPS) ** 0.5)

NC = 2
NS = 16
NW = NC * NS
EW = E // NW
CHUNK = 80
EWP = 10240
NCHUNK = EWP // CHUNK
NSLAB = NCHUNK // 8
NJUNK = N
WSTEP = 624
WSLAB = 640
ZCH = 80

NBLK = 10
BLK = N // NBLK


def _spmm_body(h_hbm, src1d, dst3d, out,
               sidx, didx, g0, g1,
               gsem0, gsem1, ssem0, ssem1, isem, aggsh):
    c = lax.axis_index("c")
    s = lax.axis_index("s")
    wid = c * NS + s

    def zrow(i, carry):
        for j in range(8):
            g0[i, pl.ds(j * 16, 16)] = jnp.zeros((16,), jnp.float32)
        return carry
    lax.fori_loop(0, ZCH, zrow, 0)
    for t in range(WSLAB // ZCH):
        pltpu.sync_copy(g0.at[pl.ds(0, ZCH)],
                        aggsh.at[pl.ds(s * WSTEP + t * ZCH, ZCH)])

    pltpu.sync_copy(src1d.at[pl.ds(wid * EWP, EWP)], sidx)

    def istart(j):
        p = (j % 2) * 8
        pltpu.async_copy(dst3d.at[wid, pl.ds(j * 8, 8)],
                         didx.at[pl.ds(p, 8)], isem)

    def iwait():
        pltpu.make_async_copy(dst3d.at[wid, pl.ds(0, 8)],
                              didx.at[pl.ds(0, 8)], isem).wait()

    GSPLIT = 2
    GS = CHUNK // GSPLIT

    def gstart(k, buf, sem):
        for t in range(GSPLIT):
            pltpu.async_copy(
                h_hbm.at[sidx.at[pl.ds(k * CHUNK + t * GS, GS)]],
                buf.at[pl.ds(t * GS, GS)], sem)

    def gwait(k, buf, sem):
        for t in range(GSPLIT):
            pltpu.make_async_copy(
                h_hbm.at[sidx.at[pl.ds(k * CHUNK + t * GS, GS)]],
                buf.at[pl.ds(t * GS, GS)], sem).wait()

    def sstart(k, buf, sem):
        pltpu.async_copy(buf, aggsh.at[didx.at[k % 16]], sem, add=True)

    def swait(k, buf, sem):
        pltpu.make_async_copy(buf, aggsh.at[didx.at[k % 16]], sem).wait()

    istart(0)
    iwait()
    gstart(0, g0, gsem0)

    def pair(j, carry):
        k0 = 2 * j

        @pl.when(jnp.logical_and(k0 % 8 == 0, k0 < 8 * (NSLAB - 1)))
        def _():
            istart(k0 // 8 + 1)
        gwait(k0, g0, gsem0)
        sstart(k0, g0, ssem0)

        @pl.when(j > 0)
        def _():
            swait(k0 - 1, g1, ssem1)
        gstart(k0 + 1, g1, gsem1)
        gwait(k0 + 1, g1, gsem1)
        sstart(k0 + 1, g1, ssem1)
        swait(k0, g0, ssem0)

        @pl.when(jnp.logical_and(k0 % 8 == 6, k0 < 8 * (NSLAB - 1)))
        def _():
            iwait()

        @pl.when(k0 < NCHUNK - 2)
        def _():
            gstart(k0 + 2, g0, gsem0)
        return carry
    lax.fori_loop(0, NCHUNK // 2, pair, 0)
    swait(NCHUNK - 1, g1, ssem1)
    plsc.subcore_barrier()

    rs = s * WSTEP
    pltpu.sync_copy(aggsh.at[pl.ds(rs, WSLAB)], out.at[c, pl.ds(rs, WSLAB)])


@functools.cache
def _get_spmm():
    mesh = plsc.VectorSubcoreMesh(core_axis_name="c", subcore_axis_name="s",
                                  num_cores=NC, num_subcores=NS)
    return pl.kernel(
        _spmm_body,
        out_type=jax.ShapeDtypeStruct((NC, N, D), jnp.float32),
        mesh=mesh,
        scratch_types=[
            pltpu.VMEM((EWP,), jnp.int32),
            pltpu.VMEM((16, CHUNK), jnp.int32),
            pltpu.VMEM((CHUNK, D), jnp.float32),
            pltpu.VMEM((CHUNK, D), jnp.float32),
            pltpu.SemaphoreType.DMA,
            pltpu.SemaphoreType.DMA,
            pltpu.SemaphoreType.DMA,
            pltpu.SemaphoreType.DMA,
            pltpu.SemaphoreType.DMA,
            pltpu.VMEM_SHARED((N + 8, D), jnp.float32),
        ],
    )


def _deg_body(dst3d, out, didx, obuf, ssem, degsh):
    c = lax.axis_index("c")
    s = lax.axis_index("s")
    wid = c * NS + s

    def zrow(i, carry):
        for j in range(8):
            obuf[i, pl.ds(j * 16, 16)] = jnp.zeros((16,), jnp.float32)
        return carry
    lax.fori_loop(0, ZCH, zrow, 0)
    for t in range(WSLAB // ZCH):
        pltpu.sync_copy(obuf.at[pl.ds(0, ZCH)],
                        degsh.at[pl.ds(s * WSTEP + t * ZCH, ZCH)])

    def fill(i, carry):
        for j in range(8):
            obuf[i, pl.ds(j * 16, 16)] = jnp.ones((16,), jnp.float32)
        return carry
    lax.fori_loop(0, CHUNK, fill, 0)
    pltpu.sync_copy(dst3d.at[wid], didx)
    plsc.subcore_barrier()

    DEPTH = 8

    def step(k, carry):
        pltpu.async_copy(obuf, degsh.at[didx.at[k]], ssem, add=True)

        @pl.when(k >= DEPTH)
        def _():
            pltpu.make_async_copy(obuf, degsh.at[didx.at[k]], ssem).wait()
        return carry
    lax.fori_loop(0, NCHUNK, step, 0)

    def drain(k, carry):
        pltpu.make_async_copy(obuf, degsh.at[didx.at[0]], ssem).wait()
        return carry
    lax.fori_loop(0, DEPTH, drain, 0)
    plsc.subcore_barrier()

    rs = s * WSTEP
    pltpu.sync_copy(degsh.at[pl.ds(rs, WSLAB)], out.at[c, pl.ds(rs, WSLAB)])


@functools.cache
def _get_deg():
    mesh = plsc.VectorSubcoreMesh(core_axis_name="c", subcore_axis_name="s",
                                  num_cores=NC, num_subcores=NS)
    return pl.kernel(
        _deg_body,
        out_type=jax.ShapeDtypeStruct((NC, N, D), jnp.float32),
        mesh=mesh,
        scratch_types=[
            pltpu.VMEM((NCHUNK, CHUNK), jnp.int32),
            pltpu.VMEM((CHUNK, D), jnp.float32),
            pltpu.SemaphoreType.DMA,
            pltpu.VMEM_SHARED((N + 8, D), jnp.float32),
        ],
    )


def _bn_body(x, g, b, o):
    o[...] = x[...] * g[...] + b[...]


_bn = pl.pallas_call(
    _bn_body,
    out_shape=jax.ShapeDtypeStruct((N, D), jnp.float32),
    grid=(NBLK,),
    in_specs=[
        pl.BlockSpec((BLK, D), lambda i: (i, 0)),
        pl.BlockSpec((1, D), lambda i: (0, 0)),
        pl.BlockSpec((1, D), lambda i: (0, 0)),
    ],
    out_specs=pl.BlockSpec((BLK, D), lambda i: (i, 0)),
)


def _dense_body(ap, h, r, wl, bl, wr, o):
    a = (ap[0] + ap[1]) * r[...]
    acc = jnp.dot(a, wl[...], preferred_element_type=jnp.float32)
    acc += jnp.dot(h[...], wr[...], preferred_element_type=jnp.float32)
    o[...] = jnp.maximum(acc + bl[...], 0.0)


_dense = pl.pallas_call(
    _dense_body,
    out_shape=jax.ShapeDtypeStruct((N, H), jnp.float32),
    grid=(NBLK,),
    in_specs=[
        pl.BlockSpec((NC, BLK, H), lambda i: (0, i, 0)),
        pl.BlockSpec((BLK, H), lambda i: (i, 0)),
        pl.BlockSpec((BLK, 1), lambda i: (i, 0)),
        pl.BlockSpec((H, H), lambda i: (0, 0)),
        pl.BlockSpec((1, H), lambda i: (0, 0)),
        pl.BlockSpec((H, H), lambda i: (0, 0)),
    ],
    out_specs=pl.BlockSpec((BLK, H), lambda i: (i, 0)),
)


def _pool_body(h, brow, bcol, pg, pb, w1, b1, w2, b2, w3, b3, o,
               ssum, smax, scnt):
    i = pl.program_id(0)

    @pl.when(i == 0)
    def _():
        ssum[...] = jnp.zeros_like(ssum)
        smax[...] = jnp.full_like(smax, -jnp.inf)
        scnt[...] = jnp.zeros_like(scnt)

    hb = h[...]
    br = brow[0]
    giota = lax.broadcasted_iota(jnp.int32, (G, 1), 0)
    onehot = jnp.where(br == giota, 1.0, 0.0)
    ssum[...] += jnp.dot(onehot, hb, preferred_element_type=jnp.float32)
    scnt[...] += jnp.sum(onehot, axis=1, keepdims=True)

    bc = bcol[...]
    g0 = jnp.min(br)
    g1 = jnp.max(br)

    def mstep(g, carry):
        m = jnp.max(jnp.where(bc == g, hb, -jnp.inf), axis=0, keepdims=True)
        smax[pl.ds(g, 1), :] = jnp.maximum(smax[pl.ds(g, 1), :], m)
        return carry
    lax.fori_loop(g0, g1 + 1, mstep, 0)

    @pl.when(i == NBLK - 1)
    def _():
        cnt = scnt[...]
        mean = ssum[...] / jnp.maximum(cnt, 1.0)
        mx = smax[...]
        mx = jnp.where(mx == -jnp.inf, 0.0, mx)
        p = jnp.concatenate([mean, mx], axis=1)
        p = p * (pg[...] * ISQ) + pb[...]
        p = jnp.maximum(jnp.dot(p, w1[...], preferred_element_type=jnp.float32) + b1[...], 0.0)
        p = jnp.maximum(jnp.dot(p, w2[...], preferred_element_type=jnp.float32) + b2[...], 0.0)
        o[...] = (jnp.dot(p, w3[...], preferred_element_type=jnp.float32) + b3[...]) * (1.0 / TEMP)


def _make_pool(m1, m2, out_dim):
    return pl.pallas_call(
        _pool_body,
        out_shape=jax.ShapeDtypeStruct((G, out_dim), jnp.float32),
        grid=(NBLK,),
        in_specs=[
            pl.BlockSpec((BLK, H), lambda i: (i, 0)),
            pl.BlockSpec((1, 1, BLK), lambda i: (i, 0, 0)),
            pl.BlockSpec((BLK, 1), lambda i: (i, 0)),
            pl.BlockSpec((1, 2 * H), lambda i: (0, 0)),
            pl.BlockSpec((1, 2 * H), lambda i: (0, 0)),
            pl.BlockSpec((2 * H, m1), lambda i: (0, 0)),
            pl.BlockSpec((1, m1), lambda i: (0, 0)),
            pl.BlockSpec((m1, m2), lambda i: (0, 0)),
            pl.BlockSpec((1, m2), lambda i: (0, 0)),
            pl.BlockSpec((m2, out_dim), lambda i: (0, 0)),
            pl.BlockSpec((1, out_dim), lambda i: (0, 0)),
        ],
        out_specs=pl.BlockSpec((G, out_dim), lambda i: (0, 0)),
        scratch_shapes=[
            pltpu.VMEM((G, H), jnp.float32),
            pltpu.VMEM((G, H), jnp.float32),
            pltpu.VMEM((G, 1), jnp.float32),
        ],
    )


def kernel(x, edge_index, batch, bn_g, bn_b,
           Wl1, bl1, Wr1, Wl2, bl2, Wr2, Wl3, bl3, Wr3, Wl4, bl4, Wr4,
           pbn_g, pbn_b, W1, b1, W2, b2, W3, b3):
    pad_s = jnp.zeros((NW, EWP - EW), jnp.int32)
    pad_d = jnp.full((NW, EWP - EW), NJUNK, jnp.int32)
    src1d = jnp.concatenate(
        [edge_index[0].reshape(NW, EW), pad_s], axis=1).reshape(NW * EWP)
    dst3d = jnp.concatenate(
        [edge_index[1].reshape(NW, EW), pad_d], axis=1).reshape(NW, NCHUNK, CHUNK)

    h = _bn(x, (bn_g * ISQ).reshape(1, D), bn_b.reshape(1, D))

    dp = _get_deg()(dst3d)
    recip = 1.0 / jnp.maximum(dp[0, :, :1] + dp[1, :, :1], 1.0)


    for Wl, bl, Wr in ((Wl1, bl1, Wr1), (Wl2, bl2, Wr2),
                       (Wl3, bl3, Wr3), (Wl4, bl4, Wr4)):
        ap = _get_spmm()(h, src1d, dst3d)
        h = _dense(ap, h, recip, Wl, bl.reshape(1, H), Wr)

    m1 = W1.shape[1]
    m2 = W2.shape[1]
    out_dim = W3.shape[1]
    pool = _make_pool(m1, m2, out_dim)
    return pool(h, batch.reshape(NBLK, 1, BLK), batch.reshape(N, 1),
                pbn_g.reshape(1, 2 * H), pbn_b.reshape(1, 2 * H),
                W1, b1.reshape(1, m1), W2, b2.reshape(1, m2),
                W3, b3.reshape(1, out_dim))


# speedup vs baseline: 1.3700x; 1.3700x over previous
import functools

import jax
import jax.numpy as jnp
from jax import lax
from jax.experimental import pallas as pl
from jax.experimental.pallas import tpu as pltpu
from jax.experimental.pallas import tpu_sc as plsc

N = 10000
E = 320000
D = 128
H = 128
G = 64
TEMP = 2.0
EPS = 1e-5
ISQ = float(1.0 / (1.0 + EPS) ** 0.5)

NC = 2
NS = 16
NW = NC * NS
EW = E // NW
CHUNK = 80
NCHUNK = EW // CHUNK
DCHUNK = 128
EWP = 10240
DNCHUNK = EWP // DCHUNK
NJUNK = N
WSTEP = 624
WSLAB = 640
ZCH = 80

NBLK = 10
BLK = N // NBLK


def _spmm_body(h_hbm, src1d, dst1d, out,
               sidx, didx, gbuf, gsem, aggsh):
    c = lax.axis_index("c")
    s = lax.axis_index("s")

    def zrow(i, carry):
        for j in range(8):
            gbuf[i, pl.ds(j * 16, 16)] = jnp.zeros((16,), jnp.float32)
        return carry
    lax.fori_loop(0, ZCH, zrow, 0)
    for t in range(WSLAB // ZCH):
        pltpu.sync_copy(gbuf.at[pl.ds(0, ZCH)],
                        aggsh.at[pl.ds(s * WSTEP + t * ZCH, ZCH)])
    plsc.subcore_barrier()

    base = (c * NS + s) * EW

    def step(k, carry):
        off = base + k * CHUNK
        pltpu.sync_copy(src1d.at[pl.ds(off, CHUNK)], sidx)
        pltpu.sync_copy(dst1d.at[pl.ds(off, CHUNK)], didx)
        pltpu.async_copy(h_hbm.at[sidx], gbuf, gsem).wait()
        pltpu.sync_copy(gbuf, aggsh.at[didx], add=True)
        return carry
    lax.fori_loop(0, NCHUNK, step, 0)
    plsc.subcore_barrier()

    rs = s * WSTEP
    pltpu.sync_copy(aggsh.at[pl.ds(rs, WSLAB)], out.at[c, pl.ds(rs, WSLAB)])


@functools.cache
def _get_spmm():
    mesh = plsc.VectorSubcoreMesh(core_axis_name="c", subcore_axis_name="s",
                                  num_cores=NC, num_subcores=NS)
    return pl.kernel(
        _spmm_body,
        out_type=jax.ShapeDtypeStruct((NC, N, D), jnp.float32),
        mesh=mesh,
        scratch_types=[
            pltpu.VMEM((CHUNK,), jnp.int32),
            pltpu.VMEM((CHUNK,), jnp.int32),
            pltpu.VMEM((CHUNK, D), jnp.float32),
            pltpu.SemaphoreType.DMA,
            pltpu.VMEM_SHARED((N, D), jnp.float32),
        ],
    )


def _deg_body(dst3d, out, didx, obuf, ssem, degsh):
    c = lax.axis_index("c")
    s = lax.axis_index("s")
    wid = c * NS + s

    def zrow(i, carry):
        for j in range(8):
            obuf[i, pl.ds(j * 16, 16)] = jnp.zeros((16,), jnp.float32)
        return carry
    lax.fori_loop(0, ZCH, zrow, 0)
    for t in range(WSLAB // ZCH):
        pltpu.sync_copy(obuf.at[pl.ds(0, ZCH)],
                        degsh.at[pl.ds(s * WSTEP + t * ZCH, ZCH)])

    def fill(i, carry):
        for j in range(8):
            obuf[i, pl.ds(j * 16, 16)] = jnp.ones((16,), jnp.float32)
        return carry
    lax.fori_loop(0, DCHUNK, fill, 0)
    pltpu.sync_copy(dst3d.at[wid], didx)
    plsc.subcore_barrier()

    DEPTH = 8

    def step(k, carry):
        pltpu.async_copy(obuf, degsh.at[didx.at[k]], ssem, add=True)

        @pl.when(k >= DEPTH)
        def _():
            pltpu.make_async_copy(obuf, degsh.at[didx.at[k]], ssem).wait()
        return carry
    lax.fori_loop(0, DNCHUNK, step, 0)

    def drain(k, carry):
        pltpu.make_async_copy(obuf, degsh.at[didx.at[0]], ssem).wait()
        return carry
    lax.fori_loop(0, DEPTH, drain, 0)
    plsc.subcore_barrier()

    rs = s * WSTEP
    pltpu.sync_copy(degsh.at[pl.ds(rs, WSLAB)], out.at[c, pl.ds(rs, WSLAB)])


@functools.cache
def _get_deg():
    mesh = plsc.VectorSubcoreMesh(core_axis_name="c", subcore_axis_name="s",
                                  num_cores=NC, num_subcores=NS)
    return pl.kernel(
        _deg_body,
        out_type=jax.ShapeDtypeStruct((NC, N, D), jnp.float32),
        mesh=mesh,
        scratch_types=[
            pltpu.VMEM((DNCHUNK, DCHUNK), jnp.int32),
            pltpu.VMEM((DCHUNK, D), jnp.float32),
            pltpu.SemaphoreType.DMA,
            pltpu.VMEM_SHARED((N + 8, D), jnp.float32),
        ],
    )


def _bn_body(x, g, b, o):
    o[...] = x[...] * g[...] + b[...]


_bn = pl.pallas_call(
    _bn_body,
    out_shape=jax.ShapeDtypeStruct((N, D), jnp.float32),
    grid=(NBLK,),
    in_specs=[
        pl.BlockSpec((BLK, D), lambda i: (i, 0)),
        pl.BlockSpec((1, D), lambda i: (0, 0)),
        pl.BlockSpec((1, D), lambda i: (0, 0)),
    ],
    out_specs=pl.BlockSpec((BLK, D), lambda i: (i, 0)),
)


def _dense_body(ap, h, r, wl, bl, wr, o):
    a = (ap[0] + ap[1]) * r[...]
    acc = jnp.dot(a, wl[...], preferred_element_type=jnp.float32)
    acc += jnp.dot(h[...], wr[...], preferred_element_type=jnp.float32)
    o[...] = jnp.maximum(acc + bl[...], 0.0)


_dense = pl.pallas_call(
    _dense_body,
    out_shape=jax.ShapeDtypeStruct((N, H), jnp.float32),
    grid=(NBLK,),
    in_specs=[
        pl.BlockSpec((NC, BLK, H), lambda i: (0, i, 0)),
        pl.BlockSpec((BLK, H), lambda i: (i, 0)),
        pl.BlockSpec((BLK, 1), lambda i: (i, 0)),
        pl.BlockSpec((H, H), lambda i: (0, 0)),
        pl.BlockSpec((1, H), lambda i: (0, 0)),
        pl.BlockSpec((H, H), lambda i: (0, 0)),
    ],
    out_specs=pl.BlockSpec((BLK, H), lambda i: (i, 0)),
)


def _pool_body(h, brow, bcol, pg, pb, w1, b1, w2, b2, w3, b3, o,
               ssum, smax, scnt):
    i = pl.program_id(0)

    @pl.when(i == 0)
    def _():
        ssum[...] = jnp.zeros_like(ssum)
        smax[...] = jnp.full_like(smax, -jnp.inf)
        scnt[...] = jnp.zeros_like(scnt)

    hb = h[...]
    br = brow[0]
    giota = lax.broadcasted_iota(jnp.int32, (G, 1), 0)
    onehot = jnp.where(br == giota, 1.0, 0.0)
    ssum[...] += jnp.dot(onehot, hb, preferred_element_type=jnp.float32)
    scnt[...] += jnp.sum(onehot, axis=1, keepdims=True)

    bc = bcol[...]
    g0 = jnp.min(br)
    g1 = jnp.max(br)

    def mstep(g, carry):
        m = jnp.max(jnp.where(bc == g, hb, -jnp.inf), axis=0, keepdims=True)
        smax[pl.ds(g, 1), :] = jnp.maximum(smax[pl.ds(g, 1), :], m)
        return carry
    lax.fori_loop(g0, g1 + 1, mstep, 0)

    @pl.when(i == NBLK - 1)
    def _():
        cnt = scnt[...]
        mean = ssum[...] / jnp.maximum(cnt, 1.0)
        mx = smax[...]
        mx = jnp.where(mx == -jnp.inf, 0.0, mx)
        p = jnp.concatenate([mean, mx], axis=1)
        p = p * (pg[...] * ISQ) + pb[...]
        p = jnp.maximum(jnp.dot(p, w1[...], preferred_element_type=jnp.float32) + b1[...], 0.0)
        p = jnp.maximum(jnp.dot(p, w2[...], preferred_element_type=jnp.float32) + b2[...], 0.0)
        o[...] = (jnp.dot(p, w3[...], preferred_element_type=jnp.float32) + b3[...]) * (1.0 / TEMP)


def _make_pool(m1, m2, out_dim):
    return pl.pallas_call(
        _pool_body,
        out_shape=jax.ShapeDtypeStruct((G, out_dim), jnp.float32),
        grid=(NBLK,),
        in_specs=[
            pl.BlockSpec((BLK, H), lambda i: (i, 0)),
            pl.BlockSpec((1, 1, BLK), lambda i: (i, 0, 0)),
            pl.BlockSpec((BLK, 1), lambda i: (i, 0)),
            pl.BlockSpec((1, 2 * H), lambda i: (0, 0)),
            pl.BlockSpec((1, 2 * H), lambda i: (0, 0)),
            pl.BlockSpec((2 * H, m1), lambda i: (0, 0)),
            pl.BlockSpec((1, m1), lambda i: (0, 0)),
            pl.BlockSpec((m1, m2), lambda i: (0, 0)),
            pl.BlockSpec((1, m2), lambda i: (0, 0)),
            pl.BlockSpec((m2, out_dim), lambda i: (0, 0)),
            pl.BlockSpec((1, out_dim), lambda i: (0, 0)),
        ],
        out_specs=pl.BlockSpec((G, out_dim), lambda i: (0, 0)),
        scratch_shapes=[
            pltpu.VMEM((G, H), jnp.float32),
            pltpu.VMEM((G, H), jnp.float32),
            pltpu.VMEM((G, 1), jnp.float32),
        ],
    )


def kernel(x, edge_index, batch, bn_g, bn_b,
           Wl1, bl1, Wr1, Wl2, bl2, Wr2, Wl3, bl3, Wr3, Wl4, bl4, Wr4,
           pbn_g, pbn_b, W1, b1, W2, b2, W3, b3):
    src1d = edge_index[0]
    dst1d = edge_index[1]
    pad_d = jnp.full((NW, EWP - EW), NJUNK, jnp.int32)
    dst3d = jnp.concatenate(
        [dst1d.reshape(NW, EW), pad_d], axis=1).reshape(NW, DNCHUNK, DCHUNK)

    h = _bn(x, (bn_g * ISQ).reshape(1, D), bn_b.reshape(1, D))

    dp = _get_deg()(dst3d)
    recip = 1.0 / jnp.maximum(dp[0, :, :1] + dp[1, :, :1], 1.0)


    for Wl, bl, Wr in ((Wl1, bl1, Wr1), (Wl2, bl2, Wr2),
                       (Wl3, bl3, Wr3), (Wl4, bl4, Wr4)):
        ap = _get_spmm()(h, src1d, dst1d)
        h = _dense(ap, h, recip, Wl, bl.reshape(1, H), Wr)

    m1 = W1.shape[1]
    m2 = W2.shape[1]
    out_dim = W3.shape[1]
    pool = _make_pool(m1, m2, out_dim)
    return pool(h, batch.reshape(NBLK, 1, BLK), batch.reshape(N, 1),
                pbn_g.reshape(1, 2 * H), pbn_b.reshape(1, 2 * H),
                W1, b1.reshape(1, m1), W2, b2.reshape(1, m2),
                W3, b3.reshape(1, out_dim))


# speedup vs baseline: 2.0181x; 1.4731x over previous
import functools

import jax
import jax.numpy as jnp
from jax import lax
from jax.experimental import pallas as pl
from jax.experimental.pallas import tpu as pltpu
from jax.experimental.pallas import tpu_sc as plsc

N = 10000
E = 320000
D = 128
H = 128
G = 64
TEMP = 2.0
EPS = 1e-5
ISQ = float(1.0 / (1.0 + EPS) ** 0.5)

NC = 2
NS = 16
NW = NC * NS
EW = E // NW
CHUNK = 80
NCHUNK = EW // CHUNK
DCHUNK = 128
EWP = 10240
DNCHUNK = EWP // DCHUNK
NJUNK = N
WSTEP = 624
WSLAB = 640
ZCH = 80

NBLK = 10
BLK = N // NBLK


def _spmm_body(h_hbm, src1d, dst1d, out,
               sidx0, didx0, sidx1, didx1, g0, g1,
               gsem0, gsem1, ssem0, ssem1, aggsh):
    c = lax.axis_index("c")
    s = lax.axis_index("s")

    def zrow(i, carry):
        for j in range(8):
            g0[i, pl.ds(j * 16, 16)] = jnp.zeros((16,), jnp.float32)
        return carry
    lax.fori_loop(0, ZCH, zrow, 0)
    for t in range(WSLAB // ZCH):
        pltpu.sync_copy(g0.at[pl.ds(0, ZCH)],
                        aggsh.at[pl.ds(s * WSTEP + t * ZCH, ZCH)])
    plsc.subcore_barrier()

    base = (c * NS + s) * EW

    def icopy(k, sb, db):
        off = base + k * CHUNK
        pltpu.sync_copy(src1d.at[pl.ds(off, CHUNK)], sb)
        pltpu.sync_copy(dst1d.at[pl.ds(off, CHUNK)], db)

    def pair(j, carry):
        k0 = 2 * j

        @pl.when(j > 0)
        def _():
            pltpu.make_async_copy(g0, aggsh.at[didx0], ssem0).wait()
        icopy(k0, sidx0, didx0)
        pltpu.async_copy(h_hbm.at[sidx0], g0, gsem0)

        @pl.when(j > 0)
        def _():
            pltpu.make_async_copy(g1, aggsh.at[didx1], ssem1).wait()
        icopy(k0 + 1, sidx1, didx1)
        pltpu.async_copy(h_hbm.at[sidx1], g1, gsem1)

        pltpu.make_async_copy(h_hbm.at[sidx0], g0, gsem0).wait()
        pltpu.async_copy(g0, aggsh.at[didx0], ssem0, add=True)
        pltpu.make_async_copy(h_hbm.at[sidx1], g1, gsem1).wait()
        pltpu.async_copy(g1, aggsh.at[didx1], ssem1, add=True)
        return carry
    lax.fori_loop(0, NCHUNK // 2, pair, 0)

    pltpu.make_async_copy(g0, aggsh.at[didx0], ssem0).wait()
    pltpu.make_async_copy(g1, aggsh.at[didx1], ssem1).wait()
    icopy(NCHUNK - 1, sidx0, didx0)
    pltpu.async_copy(h_hbm.at[sidx0], g0, gsem0).wait()
    pltpu.sync_copy(g0, aggsh.at[didx0], add=True)
    plsc.subcore_barrier()

    rs = s * WSTEP
    pltpu.sync_copy(aggsh.at[pl.ds(rs, WSLAB)], out.at[c, pl.ds(rs, WSLAB)])


@functools.cache
def _get_spmm():
    mesh = plsc.VectorSubcoreMesh(core_axis_name="c", subcore_axis_name="s",
                                  num_cores=NC, num_subcores=NS)
    return pl.kernel(
        _spmm_body,
        out_type=jax.ShapeDtypeStruct((NC, N, D), jnp.float32),
        mesh=mesh,
        scratch_types=[
            pltpu.VMEM((CHUNK,), jnp.int32),
            pltpu.VMEM((CHUNK,), jnp.int32),
            pltpu.VMEM((CHUNK,), jnp.int32),
            pltpu.VMEM((CHUNK,), jnp.int32),
            pltpu.VMEM((CHUNK, D), jnp.float32),
            pltpu.VMEM((CHUNK, D), jnp.float32),
            pltpu.SemaphoreType.DMA,
            pltpu.SemaphoreType.DMA,
            pltpu.SemaphoreType.DMA,
            pltpu.SemaphoreType.DMA,
            pltpu.VMEM_SHARED((N, D), jnp.float32),
        ],
    )


def _deg_body(dst3d, out, didx, obuf, ssem, degsh):
    c = lax.axis_index("c")
    s = lax.axis_index("s")
    wid = c * NS + s

    def zrow(i, carry):
        for j in range(8):
            obuf[i, pl.ds(j * 16, 16)] = jnp.zeros((16,), jnp.float32)
        return carry
    lax.fori_loop(0, ZCH, zrow, 0)
    for t in range(WSLAB // ZCH):
        pltpu.sync_copy(obuf.at[pl.ds(0, ZCH)],
                        degsh.at[pl.ds(s * WSTEP + t * ZCH, ZCH)])

    def fill(i, carry):
        for j in range(8):
            obuf[i, pl.ds(j * 16, 16)] = jnp.ones((16,), jnp.float32)
        return carry
    lax.fori_loop(0, DCHUNK, fill, 0)
    pltpu.sync_copy(dst3d.at[wid], didx)
    plsc.subcore_barrier()

    DEPTH = 8

    def step(k, carry):
        pltpu.async_copy(obuf, degsh.at[didx.at[k]], ssem, add=True)

        @pl.when(k >= DEPTH)
        def _():
            pltpu.make_async_copy(obuf, degsh.at[didx.at[k]], ssem).wait()
        return carry
    lax.fori_loop(0, DNCHUNK, step, 0)

    def drain(k, carry):
        pltpu.make_async_copy(obuf, degsh.at[didx.at[0]], ssem).wait()
        return carry
    lax.fori_loop(0, DEPTH, drain, 0)
    plsc.subcore_barrier()

    rs = s * WSTEP
    pltpu.sync_copy(degsh.at[pl.ds(rs, WSLAB)], out.at[c, pl.ds(rs, WSLAB)])


@functools.cache
def _get_deg():
    mesh = plsc.VectorSubcoreMesh(core_axis_name="c", subcore_axis_name="s",
                                  num_cores=NC, num_subcores=NS)
    return pl.kernel(
        _deg_body,
        out_type=jax.ShapeDtypeStruct((NC, N, D), jnp.float32),
        mesh=mesh,
        scratch_types=[
            pltpu.VMEM((DNCHUNK, DCHUNK), jnp.int32),
            pltpu.VMEM((DCHUNK, D), jnp.float32),
            pltpu.SemaphoreType.DMA,
            pltpu.VMEM_SHARED((N + 8, D), jnp.float32),
        ],
    )


def _bn_body(x, g, b, o):
    o[...] = x[...] * g[...] + b[...]


_bn = pl.pallas_call(
    _bn_body,
    out_shape=jax.ShapeDtypeStruct((N, D), jnp.float32),
    grid=(NBLK,),
    in_specs=[
        pl.BlockSpec((BLK, D), lambda i: (i, 0)),
        pl.BlockSpec((1, D), lambda i: (0, 0)),
        pl.BlockSpec((1, D), lambda i: (0, 0)),
    ],
    out_specs=pl.BlockSpec((BLK, D), lambda i: (i, 0)),
)


def _dense_body(ap, h, r, wl, bl, wr, o):
    a = (ap[0] + ap[1]) * r[...]
    acc = jnp.dot(a, wl[...], preferred_element_type=jnp.float32)
    acc += jnp.dot(h[...], wr[...], preferred_element_type=jnp.float32)
    o[...] = jnp.maximum(acc + bl[...], 0.0)


_dense = pl.pallas_call(
    _dense_body,
    out_shape=jax.ShapeDtypeStruct((N, H), jnp.float32),
    grid=(NBLK,),
    in_specs=[
        pl.BlockSpec((NC, BLK, H), lambda i: (0, i, 0)),
        pl.BlockSpec((BLK, H), lambda i: (i, 0)),
        pl.BlockSpec((BLK, 1), lambda i: (i, 0)),
        pl.BlockSpec((H, H), lambda i: (0, 0)),
        pl.BlockSpec((1, H), lambda i: (0, 0)),
        pl.BlockSpec((H, H), lambda i: (0, 0)),
    ],
    out_specs=pl.BlockSpec((BLK, H), lambda i: (i, 0)),
)


def _pool_body(h, brow, bcol, pg, pb, w1, b1, w2, b2, w3, b3, o,
               ssum, smax, scnt):
    i = pl.program_id(0)

    @pl.when(i == 0)
    def _():
        ssum[...] = jnp.zeros_like(ssum)
        smax[...] = jnp.full_like(smax, -jnp.inf)
        scnt[...] = jnp.zeros_like(scnt)

    hb = h[...]
    br = brow[0]
    giota = lax.broadcasted_iota(jnp.int32, (G, 1), 0)
    onehot = jnp.where(br == giota, 1.0, 0.0)
    ssum[...] += jnp.dot(onehot, hb, preferred_element_type=jnp.float32)
    scnt[...] += jnp.sum(onehot, axis=1, keepdims=True)

    bc = bcol[...]
    g0 = jnp.min(br)
    g1 = jnp.max(br)

    def mstep(g, carry):
        m = jnp.max(jnp.where(bc == g, hb, -jnp.inf), axis=0, keepdims=True)
        smax[pl.ds(g, 1), :] = jnp.maximum(smax[pl.ds(g, 1), :], m)
        return carry
    lax.fori_loop(g0, g1 + 1, mstep, 0)

    @pl.when(i == NBLK - 1)
    def _():
        cnt = scnt[...]
        mean = ssum[...] / jnp.maximum(cnt, 1.0)
        mx = smax[...]
        mx = jnp.where(mx == -jnp.inf, 0.0, mx)
        p = jnp.concatenate([mean, mx], axis=1)
        p = p * (pg[...] * ISQ) + pb[...]
        p = jnp.maximum(jnp.dot(p, w1[...], preferred_element_type=jnp.float32) + b1[...], 0.0)
        p = jnp.maximum(jnp.dot(p, w2[...], preferred_element_type=jnp.float32) + b2[...], 0.0)
        o[...] = (jnp.dot(p, w3[...], preferred_element_type=jnp.float32) + b3[...]) * (1.0 / TEMP)


def _make_pool(m1, m2, out_dim):
    return pl.pallas_call(
        _pool_body,
        out_shape=jax.ShapeDtypeStruct((G, out_dim), jnp.float32),
        grid=(NBLK,),
        in_specs=[
            pl.BlockSpec((BLK, H), lambda i: (i, 0)),
            pl.BlockSpec((1, 1, BLK), lambda i: (i, 0, 0)),
            pl.BlockSpec((BLK, 1), lambda i: (i, 0)),
            pl.BlockSpec((1, 2 * H), lambda i: (0, 0)),
            pl.BlockSpec((1, 2 * H), lambda i: (0, 0)),
            pl.BlockSpec((2 * H, m1), lambda i: (0, 0)),
            pl.BlockSpec((1, m1), lambda i: (0, 0)),
            pl.BlockSpec((m1, m2), lambda i: (0, 0)),
            pl.BlockSpec((1, m2), lambda i: (0, 0)),
            pl.BlockSpec((m2, out_dim), lambda i: (0, 0)),
            pl.BlockSpec((1, out_dim), lambda i: (0, 0)),
        ],
        out_specs=pl.BlockSpec((G, out_dim), lambda i: (0, 0)),
        scratch_shapes=[
            pltpu.VMEM((G, H), jnp.float32),
            pltpu.VMEM((G, H), jnp.float32),
            pltpu.VMEM((G, 1), jnp.float32),
        ],
    )


def kernel(x, edge_index, batch, bn_g, bn_b,
           Wl1, bl1, Wr1, Wl2, bl2, Wr2, Wl3, bl3, Wr3, Wl4, bl4, Wr4,
           pbn_g, pbn_b, W1, b1, W2, b2, W3, b3):
    src1d = edge_index[0]
    dst1d = edge_index[1]
    pad_d = jnp.full((NW, EWP - EW), NJUNK, jnp.int32)
    dst3d = jnp.concatenate(
        [dst1d.reshape(NW, EW), pad_d], axis=1).reshape(NW, DNCHUNK, DCHUNK)

    h = _bn(x, (bn_g * ISQ).reshape(1, D), bn_b.reshape(1, D))

    dp = _get_deg()(dst3d)
    recip = 1.0 / jnp.maximum(dp[0, :, :1] + dp[1, :, :1], 1.0)


    for Wl, bl, Wr in ((Wl1, bl1, Wr1), (Wl2, bl2, Wr2),
                       (Wl3, bl3, Wr3), (Wl4, bl4, Wr4)):
        ap = _get_spmm()(h, src1d, dst1d)
        h = _dense(ap, h, recip, Wl, bl.reshape(1, H), Wr)

    m1 = W1.shape[1]
    m2 = W2.shape[1]
    out_dim = W3.shape[1]
    pool = _make_pool(m1, m2, out_dim)
    return pool(h, batch.reshape(NBLK, 1, BLK), batch.reshape(N, 1),
                pbn_g.reshape(1, 2 * H), pbn_b.reshape(1, 2 * H),
                W1, b1.reshape(1, m1), W2, b2.reshape(1, m2),
                W3, b3.reshape(1, out_dim))
